# baseline (speedup 1.0000x reference)
import jax
import jax.numpy as jnp
from jax.experimental import pallas as pl

T = 8192
H = 2048
E = 64
TB = 1024


def _router_body(x_ref, w_ref, b_ref, out_ref):
    x = x_ref[...]
    w = w_ref[...]
    lt = jax.lax.dot_general(
        w, x,
        dimension_numbers=(((1,), (1,)), ((), ())),
        preferred_element_type=jnp.float32,
    ) + b_ref[...]
    row = jax.lax.broadcasted_iota(jnp.int32, lt.shape, 0)
    m1 = jnp.max(lt, axis=0)
    i1 = jnp.min(jnp.where(lt == m1, row, E), axis=0)
    first1 = row == i1[None, :]
    masked = jnp.where(first1, -jnp.inf, lt)
    m2 = jnp.max(masked, axis=0)
    i2 = jnp.min(jnp.where(masked == m2, row, E), axis=0)
    first2 = row == i2[None, :]
    r = jnp.exp(m2 - m1)
    denom = 1.0 + r
    p1 = 1.0 / denom
    p2 = r / denom
    out_t = jnp.where(first1, p1[None, :],
                      jnp.where(first2, p2[None, :], 0.0))
    eye = (jax.lax.broadcasted_iota(jnp.int32, (E, E), 0)
           == jax.lax.broadcasted_iota(jnp.int32, (E, E), 1)
           ).astype(jnp.float32)
    out_ref[...] = jax.lax.dot_general(
        out_t, eye,
        dimension_numbers=(((0,), (0,)), ((), ())),
        preferred_element_type=jnp.float32,
    )


def kernel(hidden_states, weight, bias):
    bias2d = bias.reshape(E, 1)
    return pl.pallas_call(
        _router_body,
        grid=(T // TB,),
        in_specs=[
            pl.BlockSpec((TB, H), lambda i: (i, 0)),
            pl.BlockSpec((E, H), lambda i: (0, 0)),
            pl.BlockSpec((E, 1), lambda i: (0, 0)),
        ],
        out_specs=pl.BlockSpec((TB, E), lambda i: (i, 0)),
        out_shape=jax.ShapeDtypeStruct((T, E), jnp.float32),
    )(hidden_states, weight, bias2d)
